# baseline (speedup 1.0000x reference)
import jax
import jax.numpy as jnp
from jax import lax
from jax.experimental import pallas as pl
from jax.experimental.pallas import tpu as pltpu
from jax.experimental.pallas import tpu_sc as plsc

N = 50000
E = 800000
H = 64
HH = 32
OUT = 10
NS = 16
LW = 128
EROWS = E // LW
RPS = EROWS // NS
EHALF = EROWS // 2
RPS2 = EHALF // NS
CH = 5
NCH_BASE = RPS // CH
NCH_LAST = (EROWS - (NS - 1) * RPS) // CH
NCH2_BASE = RPS2 // CH
NCH2_LAST = (EHALF - (NS - 1) * RPS2) // CH
N_PAD = 50048
NODE_SLICE = N_PAD // NS
NP2 = N_PAD // 2

_mesh = plsc.VectorSubcoreMesh(core_axis_name="c", subcore_axis_name="s")


def _sc_mp_body(table, ei3, zeros,
                out0, out1, sidxA, didxA, sidxB, didxB, rbuf, acc,
                gsem, ssem, isem):
    c = lax.axis_index("c")
    s = lax.axis_index("s")
    r0 = s * NODE_SLICE
    base = c * EHALF + s * RPS2
    nch = jnp.where(s == NS - 1, NCH2_LAST, NCH2_BASE)
    pltpu.sync_copy(zeros.at[pl.ds(r0, NODE_SLICE)], acc.at[pl.ds(r0, NODE_SLICE)])
    plsc.subcore_barrier()

    def stage(k, sp, dp, sem):
        rb = base + k * CH
        pltpu.async_copy(ei3.at[0, pl.ds(rb, CH)], sp, sem)
        pltpu.async_copy(ei3.at[1, pl.ds(rb, CH)], dp, sem)

    def process(k, par, is_first):
        sp, dp = (sidxA, didxA) if par == 0 else (sidxB, didxB)
        nsp, ndp = (sidxB, didxB) if par == 0 else (sidxA, didxA)

        if not is_first:
            pltpu.make_async_copy(ei3.at[0, pl.ds(0, CH)], sp,
                                  isem.at[par]).wait()
            pltpu.make_async_copy(ei3.at[1, pl.ds(0, CH)], dp,
                                  isem.at[par]).wait()
        for j in range(CH):
            if not is_first:
                pltpu.make_async_copy(rbuf.at[j], acc.at[dp.at[j]],
                                      ssem.at[j]).wait()
            pltpu.async_copy(table.at[sp.at[j]], rbuf.at[j], gsem.at[j])

        @pl.when(k + 1 < nch)
        def _():
            stage(k + 1, nsp, ndp, isem.at[1 - par])

        for j in range(CH):
            pltpu.make_async_copy(table.at[sp.at[j]], rbuf.at[j],
                                  gsem.at[j]).wait()
            pltpu.async_copy(rbuf.at[j], acc.at[dp.at[j]], ssem.at[j],
                             add=True)

    stage(0, sidxA, didxA, isem.at[0])
    pltpu.make_async_copy(ei3.at[0, pl.ds(0, CH)], sidxA, isem.at[0]).wait()
    pltpu.make_async_copy(ei3.at[1, pl.ds(0, CH)], didxA, isem.at[0]).wait()
    process(0, 0, True)

    def pair(g, carry):
        process(2 * g + 1, 1, False)
        process(2 * g + 2, 0, False)
        return carry

    lax.fori_loop(0, (nch - 1) // 2, pair, 0)

    @pl.when(nch % 2 == 0)
    def _():
        process(nch - 1, 1, False)

    for j in range(CH):
        pltpu.make_async_copy(rbuf.at[j], acc.at[didxA.at[j]],
                              ssem.at[j]).wait()

    plsc.subcore_barrier()

    @pl.when(c == 0)
    def _():
        pltpu.sync_copy(acc.at[pl.ds(r0, NODE_SLICE)], out0.at[pl.ds(r0, NODE_SLICE)])

    @pl.when(c == 1)
    def _():
        pltpu.sync_copy(acc.at[pl.ds(r0, NODE_SLICE)], out1.at[pl.ds(r0, NODE_SLICE)])


_sc_mp = pl.kernel(
    _sc_mp_body,
    out_type=[jax.ShapeDtypeStruct((N_PAD, H), jnp.bfloat16)] * 2,
    mesh=_mesh,
    scratch_types=[
        pltpu.VMEM((CH, LW), jnp.int32),
        pltpu.VMEM((CH, LW), jnp.int32),
        pltpu.VMEM((CH, LW), jnp.int32),
        pltpu.VMEM((CH, LW), jnp.int32),
        pltpu.VMEM((CH, LW, H), jnp.bfloat16),
        pltpu.VMEM_SHARED((N_PAD, H), jnp.bfloat16),
        pltpu.SemaphoreType.DMA((CH,)),
        pltpu.SemaphoreType.DMA((CH,)),
        pltpu.SemaphoreType.DMA((2,)),
    ],
    compiler_params=pltpu.CompilerParams(use_tc_tiling_on_sc=False),
)


def _sc_cnt_body(ei3_a, ei3_b, ones_hbm, zeros,
                 out_a, out_b, didxA, didxB, ones_v, acc, osem, ssem, isem):
    c = lax.axis_index("c")
    s = lax.axis_index("s")
    r0 = s * NODE_SLICE
    base = s * RPS
    nch = jnp.where(s == NS - 1, NCH_LAST, NCH_BASE)
    pltpu.async_copy(ones_hbm, ones_v, osem).wait()
    pltpu.sync_copy(zeros.at[pl.ds(r0, NODE_SLICE)], acc.at[pl.ds(r0, NODE_SLICE)])
    plsc.subcore_barrier()

    def main(ei3):
        def stage(k, dp, sem):
            rb = base + k * CH
            pltpu.async_copy(ei3.at[1, pl.ds(rb, CH)], dp, sem)

        def process(k, par, is_first):
            dp = didxA if par == 0 else didxB
            ndp = didxB if par == 0 else didxA
            if not is_first:
                pltpu.make_async_copy(ei3.at[1, pl.ds(0, CH)], dp,
                                      isem.at[par]).wait()
            for j in range(CH):
                pltpu.async_copy(ones_v, acc.at[dp.at[j]],
                                 ssem.at[par * CH + j], add=True)
            if not is_first:
                for j in range(CH):
                    pltpu.make_async_copy(ones_v, acc.at[ndp.at[j]],
                                          ssem.at[(1 - par) * CH + j]).wait()

            @pl.when(k + 1 < nch)
            def _():
                stage(k + 1, ndp, isem.at[1 - par])

        stage(0, didxA, isem.at[0])
        pltpu.make_async_copy(ei3.at[1, pl.ds(0, CH)], didxA, isem.at[0]).wait()
        process(0, 0, True)

        def pair(g, carry):
            process(2 * g + 1, 1, False)
            process(2 * g + 2, 0, False)
            return carry

        lax.fori_loop(0, (nch - 1) // 2, pair, 0)
        process(nch - 1, 1, False)
        for j in range(CH):
            pltpu.make_async_copy(ones_v, acc.at[didxB.at[j]],
                                  ssem.at[CH + j]).wait()

    @pl.when(c == 0)
    def _():
        main(ei3_a)

    @pl.when(c == 1)
    def _():
        main(ei3_b)

    plsc.subcore_barrier()

    @pl.when(c == 0)
    def _():
        pltpu.sync_copy(acc.at[pl.ds(r0, NODE_SLICE)], out_a.at[pl.ds(r0, NODE_SLICE)])

    @pl.when(c == 1)
    def _():
        pltpu.sync_copy(acc.at[pl.ds(r0, NODE_SLICE)], out_b.at[pl.ds(r0, NODE_SLICE)])


_sc_cnt = pl.kernel(
    _sc_cnt_body,
    out_type=[jax.ShapeDtypeStruct((N_PAD, H), jnp.bfloat16)] * 2,
    mesh=_mesh,
    scratch_types=[
        pltpu.VMEM((CH, LW), jnp.int32),
        pltpu.VMEM((CH, LW), jnp.int32),
        pltpu.VMEM((LW, H), jnp.bfloat16),
        pltpu.VMEM_SHARED((N_PAD, H), jnp.bfloat16),
        pltpu.SemaphoreType.DMA,
        pltpu.SemaphoreType.DMA((2 * CH,)),
        pltpu.SemaphoreType.DMA((2,)),
    ],
    compiler_params=pltpu.CompilerParams(use_tc_tiling_on_sc=False),
)


RBP = 1024
GRID = 25


def _expand_w(Wfull):
    return jnp.kron(jnp.eye(2, dtype=Wfull.dtype), Wfull)


def _tile_bias(b):
    return jnp.tile(b, 2).reshape(1, 128)


def _proj_body(xu_ref, xi_ref, bu_ref, btu_ref, bi_ref, bti_ref, hu, hi):
    def proj(x, Bw, bt):
        h = jnp.maximum(x[...] @ Bw[...] + bt[...], 0.0)
        return h.astype(jnp.bfloat16)

    hu[...] = proj(xu_ref, bu_ref, btu_ref)
    hi[...] = proj(xi_ref, bi_ref, bti_ref)


def _tc_proj(xu4, xi4, Bu, btu, Bi, bti):
    xblk = pl.BlockSpec((RBP, 4), lambda i: (i, 0))
    full = lambda shp: pl.BlockSpec(shp, lambda i: (0, 0))
    oblk = pl.BlockSpec((RBP, 128), lambda i: (i, 0))
    return pl.pallas_call(
        _proj_body,
        grid=(GRID,),
        in_specs=[xblk, xblk, full((4, 128)), full((1, 128)),
                  full((4, 128)), full((1, 128))],
        out_specs=[oblk] * 2,
        out_shape=[jax.ShapeDtypeStruct((NP2, 128), jnp.bfloat16)] * 2,
    )(xu4, xi4, Bu, btu, Bi, bti)


def _sage_out(a0, a1, cp, hp, Am, Ar, bt):
    f32 = lambda r: r[...].astype(jnp.float32)
    agg = f32(a0) + f32(a1)
    mean = agg / jnp.maximum(f32(cp), 1.0)
    return jnp.maximum(mean @ Am[...] + f32(hp) @ Ar[...] + bt[...], 0.0)


def _layer_body(ai0, ai1, ci, hi, ami, ari, bti,
                au0, au1, cu, hu, amu, aru, btu, ni, nu):
    ni[...] = _sage_out(ai0, ai1, ci, hi, ami, ari, bti).astype(jnp.bfloat16)
    nu[...] = _sage_out(au0, au1, cu, hu, amu, aru, btu).astype(jnp.bfloat16)


def _final_body(ai0, ai1, ci, hi, ami, ari, bti,
                au0, au1, cu, hu, amu, aru, btu, sum_u, sum_i):
    i = pl.program_id(0)
    oi = _sage_out(ai0, ai1, ci, hi, ami, ari, bti)
    ou = _sage_out(au0, au1, cu, hu, amu, aru, btu)
    row = lax.broadcasted_iota(jnp.int32, (RBP, 128), 0) + i * RBP
    valid = row < (N // 2)
    oi = jnp.where(valid, oi, 0.0)
    ou = jnp.where(valid, ou, 0.0)

    @pl.when(i == 0)
    def _():
        sum_u[...] = jnp.zeros_like(sum_u)
        sum_i[...] = jnp.zeros_like(sum_i)

    sum_u[...] += jnp.sum(ou, axis=0, keepdims=True)
    sum_i[...] += jnp.sum(oi, axis=0, keepdims=True)


def _layer_specs():
    blk = pl.BlockSpec((RBP, 128), lambda i: (i, 0))
    w = pl.BlockSpec((128, 128), lambda i: (0, 0))
    b = pl.BlockSpec((1, 128), lambda i: (0, 0))
    return [blk, blk, blk, blk, w, w, b,
            blk, blk, blk, blk, w, w, b]


def _tc_layer(ai, ci, hi, wi, au, cu, hu, wu):
    oblk = pl.BlockSpec((RBP, 128), lambda i: (i, 0))
    return pl.pallas_call(
        _layer_body,
        grid=(GRID,),
        in_specs=_layer_specs(),
        out_specs=[oblk] * 2,
        out_shape=[jax.ShapeDtypeStruct((NP2, 128), jnp.bfloat16)] * 2,
    )(ai[0], ai[1], ci, hi, *wi, au[0], au[1], cu, hu, *wu)


def _tc_final(ai, ci, hi, wi, au, cu, hu, wu):
    sblk = pl.BlockSpec((1, 128), lambda i: (0, 0))
    return pl.pallas_call(
        _final_body,
        grid=(GRID,),
        in_specs=_layer_specs(),
        out_specs=[sblk, sblk],
        out_shape=[jax.ShapeDtypeStruct((1, 128), jnp.float32)] * 2,
    )(ai[0], ai[1], ci, hi, *wi, au[0], au[1], cu, hu, *wu)


def _head_body(su, si, w1, b1, w2, b2, out):
    def fold(s):
        return s[:, :H] + s[:, H:]

    pooled = jnp.concatenate([fold(su[...]), fold(si[...])], axis=1)
    hid = jnp.maximum(pooled @ w1[...] + b1[...], 0.0)
    out[...] = hid @ w2[...] + b2[...]


def _tc_head(sum_u, sum_i, Wh1, bh1, Wh2, bh2):
    return pl.pallas_call(
        _head_body,
        out_shape=jax.ShapeDtypeStruct((1, OUT), jnp.float32),
    )(sum_u, sum_i, Wh1, bh1.reshape(1, H), Wh2, bh2.reshape(1, OUT))


def _packed(a):
    return jnp.reshape(a, (NP2, 128))


def _unpacked(a):
    return jnp.reshape(a, (N_PAD, H))


def kernel(x_user, x_item, W_in_user, b_in_user, W_in_item, b_in_item,
           miss_user, miss_item,
           Wl0_ri, bl0_ri, Wr0_ri, Wl0_ru, bl0_ru, Wr0_ru,
           Wl1_ri, bl1_ri, Wr1_ri, Wl1_ru, bl1_ru, Wr1_ru,
           Wh1, bh1, Wh2, bh2,
           ei_rates, ei_rated_by):
    ei3_ri = ei_rates.reshape(2, EROWS, LW)
    ei3_ru = ei_rated_by.reshape(2, EROWS, LW)

    z64 = jnp.zeros((N_PAD, H), jnp.bfloat16)
    ones64 = jnp.ones((LW, H), jnp.bfloat16)

    def proj_w(Win, miss):
        Weff = jnp.stack([Win[0], Win[1] + miss[0]])
        return jnp.kron(jnp.eye(2, dtype=Win.dtype), Weff)

    xu4 = jnp.pad(x_user, ((0, N_PAD - N), (0, 0))).reshape(NP2, 4)
    xi4 = jnp.pad(x_item, ((0, N_PAD - N), (0, 0))).reshape(NP2, 4)
    h_u, h_i = _tc_proj(xu4, xi4,
                        proj_w(W_in_user, miss_user), _tile_bias(b_in_user),
                        proj_w(W_in_item, miss_item), _tile_bias(b_in_item))

    cnt_i, cnt_u = _sc_cnt(ei3_ri, ei3_ru, ones64, z64)
    cnt_i, cnt_u = _packed(cnt_i), _packed(cnt_u)

    w0i = (_expand_w(Wl0_ri), _expand_w(Wr0_ri), _tile_bias(bl0_ri))
    w0u = (_expand_w(Wl0_ru), _expand_w(Wr0_ru), _tile_bias(bl0_ru))
    w1i = (_expand_w(Wl1_ri), _expand_w(Wr1_ri), _tile_bias(bl1_ri))
    w1u = (_expand_w(Wl1_ru), _expand_w(Wr1_ru), _tile_bias(bl1_ru))

    def mp(h_src, ei3):
        o = _sc_mp(_unpacked(h_src), ei3, z64)
        return _packed(o[0]), _packed(o[1])

    agg_i = mp(h_u, ei3_ri)
    agg_u = mp(h_i, ei3_ru)
    h_i, h_u = _tc_layer(agg_i, cnt_i, h_i, w0i, agg_u, cnt_u, h_u, w0u)

    agg_i = mp(h_u, ei3_ri)
    agg_u = mp(h_i, ei3_ru)
    sum_u, sum_i = _tc_final(agg_i, cnt_i, h_i, w1i, agg_u, cnt_u, h_u, w1u)

    return _tc_head(sum_u, sum_i, Wh1, bh1, Wh2, bh2)


# speedup vs baseline: 1.0745x; 1.0745x over previous
import jax
import jax.numpy as jnp
from jax import lax
from jax.experimental import pallas as pl
from jax.experimental.pallas import tpu as pltpu
from jax.experimental.pallas import tpu_sc as plsc

N = 50000
E = 800000
H = 64
HH = 32
OUT = 10
NS = 16
LW = 128
EROWS = E // LW
RPS = EROWS // NS
CH = 5
NCH_BASE = RPS // CH
NCH_LAST = (EROWS - (NS - 1) * RPS) // CH
N_PAD = 50048
NODE_SLICE = N_PAD // NS
NP4 = N_PAD // 4

_mesh = plsc.VectorSubcoreMesh(core_axis_name="c", subcore_axis_name="s")


def _mp_phase(table, ei3, acc, sidxA, didxA, sidxB, didxB, rbuf,
              gsem, ssem, isem, s, nch):
    base = s * RPS

    def stage(k, sp, dp, sem):
        rb = base + k * CH
        pltpu.async_copy(ei3.at[0, pl.ds(rb, CH)], sp, sem)
        pltpu.async_copy(ei3.at[1, pl.ds(rb, CH)], dp, sem)

    def process(k, par, is_first):
        sp, dp = (sidxA, didxA) if par == 0 else (sidxB, didxB)
        nsp, ndp = (sidxB, didxB) if par == 0 else (sidxA, didxA)

        if not is_first:
            pltpu.make_async_copy(ei3.at[0, pl.ds(0, CH)], sp,
                                  isem.at[par]).wait()
            pltpu.make_async_copy(ei3.at[1, pl.ds(0, CH)], dp,
                                  isem.at[par]).wait()
        for j in range(CH):
            if not is_first:
                pltpu.make_async_copy(rbuf.at[j], acc.at[dp.at[j]],
                                      ssem.at[j]).wait()
            pltpu.async_copy(table.at[sp.at[j]], rbuf.at[j], gsem.at[j])

        @pl.when(k + 1 < nch)
        def _():
            stage(k + 1, nsp, ndp, isem.at[1 - par])

        for j in range(CH):
            pltpu.make_async_copy(table.at[sp.at[j]], rbuf.at[j],
                                  gsem.at[j]).wait()
            pltpu.async_copy(rbuf.at[j], acc.at[dp.at[j]], ssem.at[j],
                             add=True)

    stage(0, sidxA, didxA, isem.at[0])
    pltpu.make_async_copy(ei3.at[0, pl.ds(0, CH)], sidxA, isem.at[0]).wait()
    pltpu.make_async_copy(ei3.at[1, pl.ds(0, CH)], didxA, isem.at[0]).wait()
    process(0, 0, True)

    def pair(g, carry):
        process(2 * g + 1, 1, False)
        process(2 * g + 2, 0, False)
        return carry

    lax.fori_loop(0, (nch - 1) // 2, pair, 0)

    @pl.when(nch % 2 == 0)
    def _():
        process(nch - 1, 1, False)

    for j in range(CH):
        pltpu.make_async_copy(rbuf.at[j], acc.at[didxA.at[j]],
                              ssem.at[j]).wait()


def _cnt_phase(ei3, acc, didxA, didxB, ones_v, ssem, isem, s, nch):
    base = s * RPS

    def stage(k, dp, sem):
        rb = base + k * CH
        pltpu.async_copy(ei3.at[1, pl.ds(rb, CH)], dp, sem)

    def process(k, par, is_first):
        dp = didxA if par == 0 else didxB
        ndp = didxB if par == 0 else didxA
        if not is_first:
            pltpu.make_async_copy(ei3.at[1, pl.ds(0, CH)], dp,
                                  isem.at[par]).wait()
        for j in range(CH):
            pltpu.async_copy(ones_v, acc.at[dp.at[j]],
                             ssem.at[par * CH + j], add=True)
        if not is_first:
            for j in range(CH):
                pltpu.make_async_copy(ones_v, acc.at[ndp.at[j]],
                                      ssem.at[(1 - par) * CH + j]).wait()

        @pl.when(k + 1 < nch)
        def _():
            stage(k + 1, ndp, isem.at[1 - par])

    stage(0, didxA, isem.at[0])
    pltpu.make_async_copy(ei3.at[1, pl.ds(0, CH)], didxA, isem.at[0]).wait()
    process(0, 0, True)

    def pair(g, carry):
        process(2 * g + 1, 1, False)
        process(2 * g + 2, 0, False)
        return carry

    lax.fori_loop(0, (nch - 1) // 2, pair, 0)
    process(nch - 1, 1, False)
    for j in range(CH):
        pltpu.make_async_copy(ones_v, acc.at[didxA.at[j]],
                              ssem.at[CH + j]).wait()


def _sc_l0_body(tA_lo, tA_hi, tB_lo, tB_hi, eiA, eiB, zeros, ones_hbm,
                cnt_a, cnt_b, outA_lo, outA_hi, outB_lo, outB_hi,
                sidxA, didxA, sidxB, didxB, rbuf, ones_v, acc,
                gsem, ssem, isem, osem):
    c = lax.axis_index("c")
    s = lax.axis_index("s")
    r0 = s * NODE_SLICE
    nch = jnp.where(s == NS - 1, NCH_LAST, NCH_BASE)
    pltpu.async_copy(ones_hbm, ones_v, osem).wait()

    def zero_slice():
        pltpu.sync_copy(zeros.at[pl.ds(r0, NODE_SLICE)],
                        acc.at[pl.ds(r0, NODE_SLICE)])

    def emit(out_lo, out_hi):
        @pl.when(c == 0)
        def _():
            pltpu.sync_copy(acc.at[pl.ds(r0, NODE_SLICE)],
                            out_lo.at[pl.ds(r0, NODE_SLICE)])

        @pl.when(c == 1)
        def _():
            pltpu.sync_copy(acc.at[pl.ds(r0, NODE_SLICE)],
                            out_hi.at[pl.ds(r0, NODE_SLICE)])

    zero_slice()
    plsc.subcore_barrier()

    @pl.when(c == 0)
    def _():
        _cnt_phase(eiA, acc, didxA, didxB, ones_v, ssem, isem, s, nch)

    @pl.when(c == 1)
    def _():
        _cnt_phase(eiB, acc, didxA, didxB, ones_v, ssem, isem, s, nch)

    plsc.subcore_barrier()
    emit(cnt_a, cnt_b)
    zero_slice()
    plsc.subcore_barrier()

    @pl.when(c == 0)
    def _():
        _mp_phase(tA_lo, eiA, acc, sidxA, didxA, sidxB, didxB, rbuf,
                  gsem, ssem, isem, s, nch)

    @pl.when(c == 1)
    def _():
        _mp_phase(tA_hi, eiA, acc, sidxA, didxA, sidxB, didxB, rbuf,
                  gsem, ssem, isem, s, nch)

    plsc.subcore_barrier()
    emit(outA_lo, outA_hi)
    zero_slice()
    plsc.subcore_barrier()

    @pl.when(c == 0)
    def _():
        _mp_phase(tB_lo, eiB, acc, sidxA, didxA, sidxB, didxB, rbuf,
                  gsem, ssem, isem, s, nch)

    @pl.when(c == 1)
    def _():
        _mp_phase(tB_hi, eiB, acc, sidxA, didxA, sidxB, didxB, rbuf,
                  gsem, ssem, isem, s, nch)

    plsc.subcore_barrier()
    emit(outB_lo, outB_hi)


def _sc_l1_body(tA_lo, tA_hi, tB_lo, tB_hi, eiA, eiB, zeros,
                outA_lo, outA_hi, outB_lo, outB_hi,
                sidxA, didxA, sidxB, didxB, rbuf, acc,
                gsem, ssem, isem):
    c = lax.axis_index("c")
    s = lax.axis_index("s")
    r0 = s * NODE_SLICE
    nch = jnp.where(s == NS - 1, NCH_LAST, NCH_BASE)

    def zero_slice():
        pltpu.sync_copy(zeros.at[pl.ds(r0, NODE_SLICE)],
                        acc.at[pl.ds(r0, NODE_SLICE)])

    def emit(out_lo, out_hi):
        @pl.when(c == 0)
        def _():
            pltpu.sync_copy(acc.at[pl.ds(r0, NODE_SLICE)],
                            out_lo.at[pl.ds(r0, NODE_SLICE)])

        @pl.when(c == 1)
        def _():
            pltpu.sync_copy(acc.at[pl.ds(r0, NODE_SLICE)],
                            out_hi.at[pl.ds(r0, NODE_SLICE)])

    zero_slice()
    plsc.subcore_barrier()

    @pl.when(c == 0)
    def _():
        _mp_phase(tA_lo, eiA, acc, sidxA, didxA, sidxB, didxB, rbuf,
                  gsem, ssem, isem, s, nch)

    @pl.when(c == 1)
    def _():
        _mp_phase(tA_hi, eiA, acc, sidxA, didxA, sidxB, didxB, rbuf,
                  gsem, ssem, isem, s, nch)

    plsc.subcore_barrier()
    emit(outA_lo, outA_hi)
    zero_slice()
    plsc.subcore_barrier()

    @pl.when(c == 0)
    def _():
        _mp_phase(tB_lo, eiB, acc, sidxA, didxA, sidxB, didxB, rbuf,
                  gsem, ssem, isem, s, nch)

    @pl.when(c == 1)
    def _():
        _mp_phase(tB_hi, eiB, acc, sidxA, didxA, sidxB, didxB, rbuf,
                  gsem, ssem, isem, s, nch)

    plsc.subcore_barrier()
    emit(outB_lo, outB_hi)


_l1_scratch = [
    pltpu.VMEM((CH, LW), jnp.int32),
    pltpu.VMEM((CH, LW), jnp.int32),
    pltpu.VMEM((CH, LW), jnp.int32),
    pltpu.VMEM((CH, LW), jnp.int32),
    pltpu.VMEM((CH, LW, HH), jnp.float32),
    pltpu.VMEM_SHARED((N_PAD, HH), jnp.float32),
    pltpu.SemaphoreType.DMA((CH,)),
    pltpu.SemaphoreType.DMA((2 * CH,)),
    pltpu.SemaphoreType.DMA((2,)),
]

_sc_l0 = pl.kernel(
    _sc_l0_body,
    out_type=[jax.ShapeDtypeStruct((N_PAD, HH), jnp.float32)] * 6,
    mesh=_mesh,
    scratch_types=_l1_scratch[:5] + [pltpu.VMEM((LW, HH), jnp.float32)]
    + _l1_scratch[5:] + [pltpu.SemaphoreType.DMA],
    compiler_params=pltpu.CompilerParams(use_tc_tiling_on_sc=False),
)

_sc_l1 = pl.kernel(
    _sc_l1_body,
    out_type=[jax.ShapeDtypeStruct((N_PAD, HH), jnp.float32)] * 4,
    mesh=_mesh,
    scratch_types=_l1_scratch,
    compiler_params=pltpu.CompilerParams(use_tc_tiling_on_sc=False),
)


RBP = 512
GRID = 25


def _expand_w(Wfull):
    Wb = Wfull.reshape(2, 32, 2, 32)
    eye4 = jnp.eye(4, dtype=Wfull.dtype)
    return jnp.einsum("ab,pkqj->pakqbj", eye4, Wb).reshape(256, 256)


def _tile_bias(b):
    return jnp.concatenate(
        [jnp.tile(b[:HH], 4), jnp.tile(b[HH:], 4)]).reshape(1, 256)


def _proj_body(xu_ref, xi_ref, bu_ref, btu_ref, bi_ref, bti_ref,
               hu_lo, hu_hi, hi_lo, hi_hi):
    def proj(x, Bw, bt):
        h = jnp.maximum(x[...] @ Bw[...] + bt[...], 0.0)
        return h[:, :128], h[:, 128:]

    hu_lo[...], hu_hi[...] = proj(xu_ref, bu_ref, btu_ref)
    hi_lo[...], hi_hi[...] = proj(xi_ref, bi_ref, bti_ref)


def _tc_proj(xu8, xi8, Bu, btu, Bi, bti):
    xblk = pl.BlockSpec((RBP, 8), lambda i: (i, 0))
    full = lambda shp: pl.BlockSpec(shp, lambda i: (0, 0))
    oblk = pl.BlockSpec((RBP, 128), lambda i: (i, 0))
    return pl.pallas_call(
        _proj_body,
        grid=(GRID,),
        in_specs=[xblk, xblk, full((8, 256)), full((1, 256)),
                  full((8, 256)), full((1, 256))],
        out_specs=[oblk] * 4,
        out_shape=[jax.ShapeDtypeStruct((NP4, 128), jnp.float32)] * 4,
    )(xu8, xi8, Bu, btu, Bi, bti)


def _sage_out(alo, ahi, cp, hlo, hhi, Am, Ar, bt):
    inv = 1.0 / jnp.maximum(cp[...], 1.0)
    mcat = jnp.concatenate([alo[...] * inv, ahi[...] * inv], axis=1)
    hcat = jnp.concatenate([hlo[...], hhi[...]], axis=1)
    return jnp.maximum(mcat @ Am[...] + hcat @ Ar[...] + bt[...], 0.0)


def _layer_body(ai_lo, ai_hi, ci, hi_lo, hi_hi, ami, ari, bti,
                au_lo, au_hi, cu, hu_lo, hu_hi, amu, aru, btu,
                ni_lo, ni_hi, nu_lo, nu_hi):
    oi = _sage_out(ai_lo, ai_hi, ci, hi_lo, hi_hi, ami, ari, bti)
    ou = _sage_out(au_lo, au_hi, cu, hu_lo, hu_hi, amu, aru, btu)
    ni_lo[...] = oi[:, :128]
    ni_hi[...] = oi[:, 128:]
    nu_lo[...] = ou[:, :128]
    nu_hi[...] = ou[:, 128:]


def _final_body(ai_lo, ai_hi, ci, hi_lo, hi_hi, ami, ari, bti,
                au_lo, au_hi, cu, hu_lo, hu_hi, amu, aru, btu,
                sum_u, sum_i):
    i = pl.program_id(0)
    oi = _sage_out(ai_lo, ai_hi, ci, hi_lo, hi_hi, ami, ari, bti)
    ou = _sage_out(au_lo, au_hi, cu, hu_lo, hu_hi, amu, aru, btu)
    row = lax.broadcasted_iota(jnp.int32, (RBP, 256), 0) + i * RBP
    valid = row < (N // 4)
    oi = jnp.where(valid, oi, 0.0)
    ou = jnp.where(valid, ou, 0.0)

    @pl.when(i == 0)
    def _():
        sum_u[...] = jnp.zeros_like(sum_u)
        sum_i[...] = jnp.zeros_like(sum_i)

    sum_u[...] += jnp.sum(ou, axis=0, keepdims=True)
    sum_i[...] += jnp.sum(oi, axis=0, keepdims=True)


def _layer_specs():
    blk = pl.BlockSpec((RBP, 128), lambda i: (i, 0))
    w = pl.BlockSpec((256, 256), lambda i: (0, 0))
    b = pl.BlockSpec((1, 256), lambda i: (0, 0))
    return [blk, blk, blk, blk, blk, w, w, b,
            blk, blk, blk, blk, blk, w, w, b]


def _tc_layer(ai, ci, hi, wi, au, cu, hu, wu):
    oblk = pl.BlockSpec((RBP, 128), lambda i: (i, 0))
    return pl.pallas_call(
        _layer_body,
        grid=(GRID,),
        in_specs=_layer_specs(),
        out_specs=[oblk] * 4,
        out_shape=[jax.ShapeDtypeStruct((NP4, 128), jnp.float32)] * 4,
    )(ai[0], ai[1], ci, hi[0], hi[1], *wi,
      au[0], au[1], cu, hu[0], hu[1], *wu)


def _tc_final(ai, ci, hi, wi, au, cu, hu, wu):
    sblk = pl.BlockSpec((1, 256), lambda i: (0, 0))
    return pl.pallas_call(
        _final_body,
        grid=(GRID,),
        in_specs=_layer_specs(),
        out_specs=[sblk, sblk],
        out_shape=[jax.ShapeDtypeStruct((1, 256), jnp.float32)] * 2,
    )(ai[0], ai[1], ci, hi[0], hi[1], *wi,
      au[0], au[1], cu, hu[0], hu[1], *wu)


def _head_body(su, si, w1, b1, w2, b2, out):
    def fold(s):
        lo = s[:, 0:32] + s[:, 32:64] + s[:, 64:96] + s[:, 96:128]
        hi = s[:, 128:160] + s[:, 160:192] + s[:, 192:224] + s[:, 224:256]
        return jnp.concatenate([lo, hi], axis=1)

    pooled = jnp.concatenate([fold(su[...]), fold(si[...])], axis=1)
    hid = jnp.maximum(pooled @ w1[...] + b1[...], 0.0)
    out[...] = hid @ w2[...] + b2[...]


def _tc_head(sum_u, sum_i, Wh1, bh1, Wh2, bh2):
    return pl.pallas_call(
        _head_body,
        out_shape=jax.ShapeDtypeStruct((1, OUT), jnp.float32),
    )(sum_u, sum_i, Wh1, bh1.reshape(1, H), Wh2, bh2.reshape(1, OUT))


def _packed(a):
    return jnp.reshape(a, (NP4, 128))


def _unpacked(a):
    return jnp.reshape(a, (N_PAD, HH))


def kernel(x_user, x_item, W_in_user, b_in_user, W_in_item, b_in_item,
           miss_user, miss_item,
           Wl0_ri, bl0_ri, Wr0_ri, Wl0_ru, bl0_ru, Wr0_ru,
           Wl1_ri, bl1_ri, Wr1_ri, Wl1_ru, bl1_ru, Wr1_ru,
           Wh1, bh1, Wh2, bh2,
           ei_rates, ei_rated_by):
    ei3_ri = ei_rates.reshape(2, EROWS, LW)
    ei3_ru = ei_rated_by.reshape(2, EROWS, LW)

    z32 = jnp.zeros((N_PAD, HH), jnp.float32)
    ones32 = jnp.ones((LW, HH), jnp.float32)

    def proj_w(Win, miss):
        Weff = jnp.stack([Win[0], Win[1] + miss[0]])
        Wb = Weff.reshape(2, 2, 32)
        eye4 = jnp.eye(4, dtype=Win.dtype)
        return jnp.einsum("ab,dpj->adpbj", eye4, Wb).reshape(8, 256)

    xu8 = jnp.pad(x_user, ((0, N_PAD - N), (0, 0))).reshape(NP4, 8)
    xi8 = jnp.pad(x_item, ((0, N_PAD - N), (0, 0))).reshape(NP4, 8)
    hu = _tc_proj(xu8, xi8,
                  proj_w(W_in_user, miss_user), _tile_bias(b_in_user),
                  proj_w(W_in_item, miss_item), _tile_bias(b_in_item))
    h_u, h_i = (hu[0], hu[1]), (hu[2], hu[3])

    w0i = (_expand_w(Wl0_ri), _expand_w(Wr0_ri), _tile_bias(bl0_ri))
    w0u = (_expand_w(Wl0_ru), _expand_w(Wr0_ru), _tile_bias(bl0_ru))
    w1i = (_expand_w(Wl1_ri), _expand_w(Wr1_ri), _tile_bias(bl1_ri))
    w1u = (_expand_w(Wl1_ru), _expand_w(Wr1_ru), _tile_bias(bl1_ru))

    l0 = _sc_l0(_unpacked(h_u[0]), _unpacked(h_u[1]),
                _unpacked(h_i[0]), _unpacked(h_i[1]),
                ei3_ri, ei3_ru, z32, ones32)
    cnt_i, cnt_u = _packed(l0[0]), _packed(l0[1])
    agg_i = (_packed(l0[2]), _packed(l0[3]))
    agg_u = (_packed(l0[4]), _packed(l0[5]))
    nh = _tc_layer(agg_i, cnt_i, h_i, w0i, agg_u, cnt_u, h_u, w0u)
    h_i, h_u = (nh[0], nh[1]), (nh[2], nh[3])

    l1 = _sc_l1(_unpacked(h_u[0]), _unpacked(h_u[1]),
                _unpacked(h_i[0]), _unpacked(h_i[1]),
                ei3_ri, ei3_ru, z32)
    agg_i = (_packed(l1[0]), _packed(l1[1]))
    agg_u = (_packed(l1[2]), _packed(l1[3]))
    sum_u, sum_i = _tc_final(agg_i, cnt_i, h_i, w1i, agg_u, cnt_u, h_u, w1u)

    return _tc_head(sum_u, sum_i, Wh1, bh1, Wh2, bh2)


# speedup vs baseline: 1.1588x; 1.0784x over previous
import jax
import jax.numpy as jnp
from jax import lax
from jax.experimental import pallas as pl
from jax.experimental.pallas import tpu as pltpu
from jax.experimental.pallas import tpu_sc as plsc

N = 50000
E = 800000
H = 64
HH = 32
OUT = 10
NS = 16
LW = 128
EROWS = E // LW
RPS = EROWS // NS
CH = 5
NCH_BASE = RPS // CH
NCH_LAST = (EROWS - (NS - 1) * RPS) // CH
N_PAD = 50048
NODE_SLICE = N_PAD // NS
NP4 = N_PAD // 4

_mesh = plsc.VectorSubcoreMesh(core_axis_name="c", subcore_axis_name="s")


def _mp_phase(table, ei3, acc, sidxA, didxA, sidxB, didxB, rbuf,
              gsem, ssem, isem, s, nch):
    base = s * RPS

    def stage(k, sp, dp, sem):
        rb = base + k * CH
        pltpu.async_copy(ei3.at[0, pl.ds(rb, CH)], sp, sem)
        pltpu.async_copy(ei3.at[1, pl.ds(rb, CH)], dp, sem)

    def process(k, par, is_first):
        sp, dp = (sidxA, didxA) if par == 0 else (sidxB, didxB)
        nsp, ndp = (sidxB, didxB) if par == 0 else (sidxA, didxA)

        if not is_first:
            pltpu.make_async_copy(ei3.at[0, pl.ds(0, CH)], sp,
                                  isem.at[par]).wait()
            pltpu.make_async_copy(ei3.at[1, pl.ds(0, CH)], dp,
                                  isem.at[par]).wait()
        for j in range(CH):
            if not is_first:
                pltpu.make_async_copy(rbuf.at[j], acc.at[dp.at[j]],
                                      ssem.at[j]).wait()
            pltpu.async_copy(table.at[sp.at[j]], rbuf.at[j], gsem.at[j])

        @pl.when(k + 1 < nch)
        def _():
            stage(k + 1, nsp, ndp, isem.at[1 - par])

        for j in range(CH):
            pltpu.make_async_copy(table.at[sp.at[j]], rbuf.at[j],
                                  gsem.at[j]).wait()
            pltpu.async_copy(rbuf.at[j], acc.at[dp.at[j]], ssem.at[j],
                             add=True)

    stage(0, sidxA, didxA, isem.at[0])
    pltpu.make_async_copy(ei3.at[0, pl.ds(0, CH)], sidxA, isem.at[0]).wait()
    pltpu.make_async_copy(ei3.at[1, pl.ds(0, CH)], didxA, isem.at[0]).wait()
    process(0, 0, True)

    def pair(g, carry):
        process(2 * g + 1, 1, False)
        process(2 * g + 2, 0, False)
        return carry

    lax.fori_loop(0, (nch - 1) // 2, pair, 0)

    @pl.when(nch % 2 == 0)
    def _():
        process(nch - 1, 1, False)

    for j in range(CH):
        pltpu.make_async_copy(rbuf.at[j], acc.at[didxA.at[j]],
                              ssem.at[j]).wait()


def _cnt_phase(ei3, acc, didxA, didxB, ones_v, ssem, isem, s, nch):
    base = s * RPS

    def stage(k, dp, sem):
        rb = base + k * CH
        pltpu.async_copy(ei3.at[1, pl.ds(rb, CH)], dp, sem)

    def process(k, par, is_first):
        dp = didxA if par == 0 else didxB
        ndp = didxB if par == 0 else didxA
        if not is_first:
            pltpu.make_async_copy(ei3.at[1, pl.ds(0, CH)], dp,
                                  isem.at[par]).wait()
        for j in range(CH):
            pltpu.async_copy(ones_v, acc.at[dp.at[j]],
                             ssem.at[par * CH + j], add=True)
        if not is_first:
            for j in range(CH):
                pltpu.make_async_copy(ones_v, acc.at[ndp.at[j]],
                                      ssem.at[(1 - par) * CH + j]).wait()

        @pl.when(k + 1 < nch)
        def _():
            stage(k + 1, ndp, isem.at[1 - par])

    stage(0, didxA, isem.at[0])
    pltpu.make_async_copy(ei3.at[1, pl.ds(0, CH)], didxA, isem.at[0]).wait()
    process(0, 0, True)

    def pair(g, carry):
        process(2 * g + 1, 1, False)
        process(2 * g + 2, 0, False)
        return carry

    lax.fori_loop(0, (nch - 1) // 2, pair, 0)
    process(nch - 1, 1, False)
    for j in range(CH):
        pltpu.make_async_copy(ones_v, acc.at[didxA.at[j]],
                              ssem.at[CH + j]).wait()


def _sc_mp_body(table_lo, table_hi, ei3, zeros, out_lo, out_hi,
                sidxA, didxA, sidxB, didxB, rbuf, acc, gsem, ssem, isem):
    c = lax.axis_index("c")
    s = lax.axis_index("s")
    r0 = s * NODE_SLICE
    nch = jnp.where(s == NS - 1, NCH_LAST, NCH_BASE)
    pltpu.sync_copy(zeros.at[pl.ds(r0, NODE_SLICE)],
                    acc.at[pl.ds(r0, NODE_SLICE)])
    plsc.subcore_barrier()

    @pl.when(c == 0)
    def _():
        _mp_phase(table_lo, ei3, acc, sidxA, didxA, sidxB, didxB, rbuf,
                  gsem, ssem, isem, s, nch)

    @pl.when(c == 1)
    def _():
        _mp_phase(table_hi, ei3, acc, sidxA, didxA, sidxB, didxB, rbuf,
                  gsem, ssem, isem, s, nch)

    plsc.subcore_barrier()

    @pl.when(c == 0)
    def _():
        pltpu.sync_copy(acc.at[pl.ds(r0, NODE_SLICE)],
                        out_lo.at[pl.ds(r0, NODE_SLICE)])

    @pl.when(c == 1)
    def _():
        pltpu.sync_copy(acc.at[pl.ds(r0, NODE_SLICE)],
                        out_hi.at[pl.ds(r0, NODE_SLICE)])


def _sc_cnt_body(eiA, eiB, ones_hbm, zeros, out_a, out_b,
                 didxA, didxB, ones_v, acc, osem, ssem, isem):
    c = lax.axis_index("c")
    s = lax.axis_index("s")
    r0 = s * NODE_SLICE
    nch = jnp.where(s == NS - 1, NCH_LAST, NCH_BASE)
    pltpu.async_copy(ones_hbm, ones_v, osem).wait()
    pltpu.sync_copy(zeros.at[pl.ds(r0, NODE_SLICE)],
                    acc.at[pl.ds(r0, NODE_SLICE)])
    plsc.subcore_barrier()

    @pl.when(c == 0)
    def _():
        _cnt_phase(eiA, acc, didxA, didxB, ones_v, ssem, isem, s, nch)

    @pl.when(c == 1)
    def _():
        _cnt_phase(eiB, acc, didxA, didxB, ones_v, ssem, isem, s, nch)

    plsc.subcore_barrier()

    @pl.when(c == 0)
    def _():
        pltpu.sync_copy(acc.at[pl.ds(r0, NODE_SLICE)],
                        out_a.at[pl.ds(r0, NODE_SLICE)])

    @pl.when(c == 1)
    def _():
        pltpu.sync_copy(acc.at[pl.ds(r0, NODE_SLICE)],
                        out_b.at[pl.ds(r0, NODE_SLICE)])


_sc_mp = pl.kernel(
    _sc_mp_body,
    out_type=[jax.ShapeDtypeStruct((N_PAD, HH), jnp.float32)] * 2,
    mesh=_mesh,
    scratch_types=[
        pltpu.VMEM((CH, LW), jnp.int32),
        pltpu.VMEM((CH, LW), jnp.int32),
        pltpu.VMEM((CH, LW), jnp.int32),
        pltpu.VMEM((CH, LW), jnp.int32),
        pltpu.VMEM((CH, LW, HH), jnp.float32),
        pltpu.VMEM_SHARED((N_PAD, HH), jnp.float32),
        pltpu.SemaphoreType.DMA((CH,)),
        pltpu.SemaphoreType.DMA((2 * CH,)),
        pltpu.SemaphoreType.DMA((2,)),
    ],
    compiler_params=pltpu.CompilerParams(use_tc_tiling_on_sc=False),
)

_sc_cnt = pl.kernel(
    _sc_cnt_body,
    out_type=[jax.ShapeDtypeStruct((N_PAD, HH), jnp.float32)] * 2,
    mesh=_mesh,
    scratch_types=[
        pltpu.VMEM((CH, LW), jnp.int32),
        pltpu.VMEM((CH, LW), jnp.int32),
        pltpu.VMEM((LW, HH), jnp.float32),
        pltpu.VMEM_SHARED((N_PAD, HH), jnp.float32),
        pltpu.SemaphoreType.DMA,
        pltpu.SemaphoreType.DMA((2 * CH,)),
        pltpu.SemaphoreType.DMA((2,)),
    ],
    compiler_params=pltpu.CompilerParams(use_tc_tiling_on_sc=False),
)


RBP = 512
GRID = 25


def _expand_w(Wfull):
    Wb = Wfull.reshape(2, 32, 2, 32)
    eye4 = jnp.eye(4, dtype=Wfull.dtype)
    return jnp.einsum("ab,pkqj->pakqbj", eye4, Wb).reshape(256, 256)


def _tile_bias(b):
    return jnp.concatenate(
        [jnp.tile(b[:HH], 4), jnp.tile(b[HH:], 4)]).reshape(1, 256)


def _proj_body(xu_ref, xi_ref, bu_ref, btu_ref, bi_ref, bti_ref,
               hu_lo, hu_hi, hi_lo, hi_hi):
    def proj(x, Bw, bt):
        h = jnp.maximum(x[...] @ Bw[...] + bt[...], 0.0)
        return h[:, :128], h[:, 128:]

    hu_lo[...], hu_hi[...] = proj(xu_ref, bu_ref, btu_ref)
    hi_lo[...], hi_hi[...] = proj(xi_ref, bi_ref, bti_ref)


def _tc_proj(xu8, xi8, Bu, btu, Bi, bti):
    xblk = pl.BlockSpec((RBP, 8), lambda i: (i, 0))
    full = lambda shp: pl.BlockSpec(shp, lambda i: (0, 0))
    oblk = pl.BlockSpec((RBP, 128), lambda i: (i, 0))
    return pl.pallas_call(
        _proj_body,
        grid=(GRID,),
        in_specs=[xblk, xblk, full((8, 256)), full((1, 256)),
                  full((8, 256)), full((1, 256))],
        out_specs=[oblk] * 4,
        out_shape=[jax.ShapeDtypeStruct((NP4, 128), jnp.float32)] * 4,
    )(xu8, xi8, Bu, btu, Bi, bti)


def _sage_out(alo, ahi, cp, hlo, hhi, Am, Ar, bt):
    inv = 1.0 / jnp.maximum(cp[...], 1.0)
    mcat = jnp.concatenate([alo[...] * inv, ahi[...] * inv], axis=1)
    hcat = jnp.concatenate([hlo[...], hhi[...]], axis=1)
    return jnp.maximum(mcat @ Am[...] + hcat @ Ar[...] + bt[...], 0.0)


def _layer_body(ai_lo, ai_hi, ci, hi_lo, hi_hi, ami, ari, bti,
                au_lo, au_hi, cu, hu_lo, hu_hi, amu, aru, btu,
                ni_lo, ni_hi, nu_lo, nu_hi):
    oi = _sage_out(ai_lo, ai_hi, ci, hi_lo, hi_hi, ami, ari, bti)
    ou = _sage_out(au_lo, au_hi, cu, hu_lo, hu_hi, amu, aru, btu)
    ni_lo[...] = oi[:, :128]
    ni_hi[...] = oi[:, 128:]
    nu_lo[...] = ou[:, :128]
    nu_hi[...] = ou[:, 128:]


def _final_body(ai_lo, ai_hi, ci, hi_lo, hi_hi, ami, ari, bti,
                au_lo, au_hi, cu, hu_lo, hu_hi, amu, aru, btu,
                sum_u, sum_i):
    i = pl.program_id(0)
    oi = _sage_out(ai_lo, ai_hi, ci, hi_lo, hi_hi, ami, ari, bti)
    ou = _sage_out(au_lo, au_hi, cu, hu_lo, hu_hi, amu, aru, btu)
    row = lax.broadcasted_iota(jnp.int32, (RBP, 256), 0) + i * RBP
    valid = row < (N // 4)
    oi = jnp.where(valid, oi, 0.0)
    ou = jnp.where(valid, ou, 0.0)

    @pl.when(i == 0)
    def _():
        sum_u[...] = jnp.zeros_like(sum_u)
        sum_i[...] = jnp.zeros_like(sum_i)

    sum_u[...] += jnp.sum(ou, axis=0, keepdims=True)
    sum_i[...] += jnp.sum(oi, axis=0, keepdims=True)


def _layer_specs():
    blk = pl.BlockSpec((RBP, 128), lambda i: (i, 0))
    w = pl.BlockSpec((256, 256), lambda i: (0, 0))
    b = pl.BlockSpec((1, 256), lambda i: (0, 0))
    return [blk, blk, blk, blk, blk, w, w, b,
            blk, blk, blk, blk, blk, w, w, b]


def _tc_layer(ai, ci, hi, wi, au, cu, hu, wu):
    oblk = pl.BlockSpec((RBP, 128), lambda i: (i, 0))
    return pl.pallas_call(
        _layer_body,
        grid=(GRID,),
        in_specs=_layer_specs(),
        out_specs=[oblk] * 4,
        out_shape=[jax.ShapeDtypeStruct((NP4, 128), jnp.float32)] * 4,
    )(ai[0], ai[1], ci, hi[0], hi[1], *wi,
      au[0], au[1], cu, hu[0], hu[1], *wu)


def _tc_final(ai, ci, hi, wi, au, cu, hu, wu):
    sblk = pl.BlockSpec((1, 256), lambda i: (0, 0))
    return pl.pallas_call(
        _final_body,
        grid=(GRID,),
        in_specs=_layer_specs(),
        out_specs=[sblk, sblk],
        out_shape=[jax.ShapeDtypeStruct((1, 256), jnp.float32)] * 2,
    )(ai[0], ai[1], ci, hi[0], hi[1], *wi,
      au[0], au[1], cu, hu[0], hu[1], *wu)


def _head_body(su, si, w1, b1, w2, b2, out):
    def fold(s):
        lo = s[:, 0:32] + s[:, 32:64] + s[:, 64:96] + s[:, 96:128]
        hi = s[:, 128:160] + s[:, 160:192] + s[:, 192:224] + s[:, 224:256]
        return jnp.concatenate([lo, hi], axis=1)

    pooled = jnp.concatenate([fold(su[...]), fold(si[...])], axis=1)
    hid = jnp.maximum(pooled @ w1[...] + b1[...], 0.0)
    out[...] = hid @ w2[...] + b2[...]


def _tc_head(sum_u, sum_i, Wh1, bh1, Wh2, bh2):
    return pl.pallas_call(
        _head_body,
        out_shape=jax.ShapeDtypeStruct((1, OUT), jnp.float32),
    )(sum_u, sum_i, Wh1, bh1.reshape(1, H), Wh2, bh2.reshape(1, OUT))


def _packed(a):
    return jnp.reshape(a, (NP4, 128))


def _unpacked(a):
    return jnp.reshape(a, (N_PAD, HH))


def kernel(x_user, x_item, W_in_user, b_in_user, W_in_item, b_in_item,
           miss_user, miss_item,
           Wl0_ri, bl0_ri, Wr0_ri, Wl0_ru, bl0_ru, Wr0_ru,
           Wl1_ri, bl1_ri, Wr1_ri, Wl1_ru, bl1_ru, Wr1_ru,
           Wh1, bh1, Wh2, bh2,
           ei_rates, ei_rated_by):
    ei3_ri = ei_rates.reshape(2, EROWS, LW)
    ei3_ru = ei_rated_by.reshape(2, EROWS, LW)

    z32 = jnp.zeros((N_PAD, HH), jnp.float32)
    ones32 = jnp.ones((LW, HH), jnp.float32)

    def proj_w(Win, miss):
        Weff = jnp.stack([Win[0], Win[1] + miss[0]])
        Wb = Weff.reshape(2, 2, 32)
        eye4 = jnp.eye(4, dtype=Win.dtype)
        return jnp.einsum("ab,dpj->adpbj", eye4, Wb).reshape(8, 256)

    xu8 = jnp.pad(x_user, ((0, N_PAD - N), (0, 0))).reshape(NP4, 8)
    xi8 = jnp.pad(x_item, ((0, N_PAD - N), (0, 0))).reshape(NP4, 8)
    hu = _tc_proj(xu8, xi8,
                  proj_w(W_in_user, miss_user), _tile_bias(b_in_user),
                  proj_w(W_in_item, miss_item), _tile_bias(b_in_item))
    h_u, h_i = (hu[0], hu[1]), (hu[2], hu[3])

    w0i = (_expand_w(Wl0_ri), _expand_w(Wr0_ri), _tile_bias(bl0_ri))
    w0u = (_expand_w(Wl0_ru), _expand_w(Wr0_ru), _tile_bias(bl0_ru))
    w1i = (_expand_w(Wl1_ri), _expand_w(Wr1_ri), _tile_bias(bl1_ri))
    w1u = (_expand_w(Wl1_ru), _expand_w(Wr1_ru), _tile_bias(bl1_ru))

    cnt = _sc_cnt(ei3_ri, ei3_ru, ones32, z32)
    cnt_i, cnt_u = _packed(cnt[0]), _packed(cnt[1])

    def mp(h_src, ei3):
        o = _sc_mp(_unpacked(h_src[0]), _unpacked(h_src[1]), ei3, z32)
        return _packed(o[0]), _packed(o[1])

    agg_i = mp(h_u, ei3_ri)
    agg_u = mp(h_i, ei3_ru)
    nh = _tc_layer(agg_i, cnt_i, h_i, w0i, agg_u, cnt_u, h_u, w0u)
    h_i, h_u = (nh[0], nh[1]), (nh[2], nh[3])

    agg_i = mp(h_u, ei3_ri)
    agg_u = mp(h_i, ei3_ru)
    sum_u, sum_i = _tc_final(agg_i, cnt_i, h_i, w1i, agg_u, cnt_u, h_u, w1u)

    return _tc_head(sum_u, sum_i, Wh1, bh1, Wh2, bh2)
